# Initial kernel scaffold; baseline (speedup 1.0000x reference)
import jax
import jax.numpy as jnp
from jax.experimental import pallas as pl


def kernel(xyz, points, W0, b0, gamma0, beta0, W1, b1, gamma1, beta1, W2, b2, gamma2, beta2):
    raise NotImplementedError("write your pallas kernel here")


# speedup vs baseline: 9.8405x; 9.8405x over previous
import functools

import jax
import jax.numpy as jnp
import numpy as np
from jax import lax
from jax.experimental import pallas as pl
from jax.experimental.pallas import tpu as pltpu
from jax.experimental.pallas import tpu_sc as plsc

B = 4
N = 8192
S = 1024
NS = 32
D_PTS = 64
C1 = 64
C2 = 64
C3 = 128
M_TOT = B * S * NS
EPS = 1e-5
R2 = float(np.float32(0.2) * np.float32(0.2))
BIG = N


def _fps_body(xyz_ref, nxyz_ref):
    xyz = xyz_ref[...]
    lane_iota2 = lax.broadcasted_iota(jnp.int32, (B, N), 1)
    lane_iota3 = lax.broadcasted_iota(jnp.int32, (B, 1, N), 2)

    def body(i, st):
        dist, far = st
        m = (lane_iota3 == far[:, None, None]).astype(jnp.float32)
        cent = jnp.sum(xyz * m, axis=2)
        nxyz_ref[pl.ds(i, 1)] = cent[None]
        d = jnp.sum((xyz - cent[:, :, None]) ** 2, axis=1)
        dist = jnp.minimum(dist, d)
        mx = jnp.max(dist, axis=1, keepdims=True)
        far = jnp.min(jnp.where(dist == mx, lane_iota2, N), axis=1).astype(
            jnp.int32)
        return dist, far

    st = (jnp.full((B, N), 1e10, jnp.float32), jnp.zeros((B,), jnp.int32))
    lax.fori_loop(0, S, body, st)


def _fps(xyz, *, interpret=False):
    return pl.pallas_call(
        _fps_body,
        out_shape=jax.ShapeDtypeStruct((S, B, 3), jnp.float32),
        interpret=interpret,
    )(xyz)


N_CHUNK = 1024
N_CH = N // N_CHUNK

def _u_body(xyz_ref, pts_ref, wx_ref, wp_ref, b0_ref, u_ref):
    x = xyz_ref[0]
    p = pts_ref[0]
    wx = wx_ref[...]
    wp = wp_ref[...]
    u = lax.dot_general(x, wx, (((0,), (1,)), ((), ())),
                        preferred_element_type=jnp.float32)
    u = u + lax.dot_general(p, wp, (((0,), (1,)), ((), ())),
                            preferred_element_type=jnp.float32)
    u_ref[...] = u + b0_ref[...]


def _compute_u(xyz, points, wx, wp, b0r, *, interpret=False):
    return pl.pallas_call(
        _u_body,
        grid=(B, N_CH),
        in_specs=[
            pl.BlockSpec((1, 3, N_CHUNK), lambda b, c: (b, 0, c)),
            pl.BlockSpec((1, D_PTS, N_CHUNK), lambda b, c: (b, 0, c)),
            pl.BlockSpec((C1, 3), lambda b, c: (0, 0)),
            pl.BlockSpec((C1, D_PTS), lambda b, c: (0, 0)),
            pl.BlockSpec((1, C1), lambda b, c: (0, 0)),
        ],
        out_specs=pl.BlockSpec((N_CHUNK, C1), lambda b, c: (b * N_CH + c, 0)),
        out_shape=jax.ShapeDtypeStruct((B * N, C1), jnp.float32),
        interpret=interpret,
    )(xyz, points, wx, wp, b0r)


S_TILE = 128
S_CH = S // S_TILE

def _bq_body(xyz_ref, nxyz_ref, out_ref):
    b = pl.program_id(0)
    x = xyz_ref[0]
    bsel = (lax.broadcasted_iota(jnp.int32, (1, B, 1), 1) == b)
    nx = jnp.sum(jnp.where(bsel, nxyz_ref[...], 0.0), axis=1)
    sqx = jnp.sum(x * x, axis=0)[None, :]
    sqn = jnp.sum(nx * nx, axis=1)[:, None]
    dot = lax.dot_general(nx, x, (((1,), (0,)), ((), ())),
                          preferred_element_type=jnp.float32)
    d = (sqn + sqx) - 2.0 * dot
    iota = lax.broadcasted_iota(jnp.int32, (S_TILE, N), 1)
    keys = jnp.where(d <= R2, iota, BIG)
    cols = []
    first = None
    for _ in range(NS):
        m = jnp.min(keys, axis=1)
        if first is None:
            first = m
        keys = jnp.where(keys == m[:, None], BIG, keys)
        cols.append(m)
    sel = jnp.stack(cols, axis=1)
    sel = jnp.where(sel == BIG, first[:, None], sel)
    out_ref[...] = sel + b * N


def _ball_query(xyz, nxyz, *, interpret=False):
    return pl.pallas_call(
        _bq_body,
        grid=(B, S_CH),
        in_specs=[
            pl.BlockSpec((1, 3, N), lambda b, c: (b, 0, 0)),
            pl.BlockSpec((S_TILE, B, 3), lambda b, c: (c, 0, 0)),
        ],
        out_specs=pl.BlockSpec((S_TILE, NS), lambda b, c: (b * S_CH + c, 0)),
        out_shape=jax.ShapeDtypeStruct((B * S, NS), jnp.int32),
        interpret=interpret,
    )(xyz, nxyz)


GCHUNK = 128
PER_W = None

@functools.lru_cache(maxsize=1)
def _make_gather():
    nw = 32
    per_w = M_TOT // nw
    iters = per_w // GCHUNK
    mesh = plsc.VectorSubcoreMesh(core_axis_name="c", subcore_axis_name="s")

    @functools.partial(
        pl.kernel,
        mesh=mesh,
        compiler_params=pltpu.CompilerParams(use_tc_tiling_on_sc=False),
        out_type=jax.ShapeDtypeStruct((M_TOT, C1), jnp.float32),
        scratch_types=[
            pltpu.VMEM((GCHUNK,), jnp.int32),
            pltpu.VMEM((GCHUNK, C1), jnp.float32),
            pltpu.SemaphoreType.DMA,
        ],
    )
    def gather_k(table_hbm, idx_hbm, out_hbm, idx_v, rows_v, sem):
        c = lax.axis_index("c")
        s = lax.axis_index("s")
        wid = s * 2 + c
        base = wid * per_w

        def body(i, carry):
            off = base + i * GCHUNK
            pltpu.sync_copy(idx_hbm.at[pl.ds(off, GCHUNK)], idx_v)
            pltpu.async_copy(table_hbm.at[idx_v], rows_v, sem).wait()
            pltpu.sync_copy(rows_v, out_hbm.at[pl.ds(off, GCHUNK)])
            return carry

        lax.fori_loop(0, iters, body, 0)

    return gather_k


M_TILE = 4096
CT = M_TILE // NS
N_TILES = M_TOT // M_TILE
MF = float(M_TOT)

def _mlp_body(z0_ref, nxyz_ref, wx_ref, w1_ref, w2_ref,
              b1_ref, b2_ref, g0_ref, be0_ref, g1_ref, be1_ref,
              g2_ref, be2_ref, out_ref, acc_ref):
    p = pl.program_id(0)
    t = pl.program_id(1)

    @pl.when((p == 0) & (t == 0))
    def _():
        acc_ref[...] = jnp.zeros((8, 128), jnp.float32)

    bsel = (lax.broadcasted_iota(jnp.int32, (1, B, 1), 1) == t // S_CH)
    nx = jnp.sum(jnp.where(bsel, nxyz_ref[...], 0.0), axis=1)
    v = lax.dot_general(nx, wx_ref[...], (((1,), (1,)), ((), ())),
                        preferred_element_type=jnp.float32)
    r_iota = lax.broadcasted_iota(jnp.int32, (M_TILE, CT), 0) // NS
    c_iota = lax.broadcasted_iota(jnp.int32, (M_TILE, CT), 1)
    expand = (r_iota == c_iota).astype(jnp.float32)
    vexp = lax.dot_general(expand, v, (((1,), (0,)), ((), ())),
                           preferred_element_type=jnp.float32)
    z = z0_ref[...] - vexp

    @pl.when(p == 0)
    def _():
        acc_ref[0:1, :C1] += jnp.sum(z, axis=0)[None]
        acc_ref[1:2, :C1] += jnp.sum(z * z, axis=0)[None]

    m0 = acc_ref[0:1, :C1] / MF
    v0 = acc_ref[1:2, :C1] / MF - m0 * m0
    y1 = jnp.maximum((z - m0) / jnp.sqrt(v0 + EPS) * g0_ref[...]
                     + be0_ref[...], 0.0)
    z2 = lax.dot_general(y1, w1_ref[...], (((1,), (1,)), ((), ())),
                         preferred_element_type=jnp.float32) + b1_ref[...]

    @pl.when(p == 1)
    def _():
        acc_ref[2:3, :C1] += jnp.sum(z2, axis=0)[None]
        acc_ref[3:4, :C1] += jnp.sum(z2 * z2, axis=0)[None]

    m1 = acc_ref[2:3, :C1] / MF
    v1 = acc_ref[3:4, :C1] / MF - m1 * m1
    y2 = jnp.maximum((z2 - m1) / jnp.sqrt(v1 + EPS) * g1_ref[...]
                     + be1_ref[...], 0.0)
    z3 = lax.dot_general(y2, w2_ref[...], (((1,), (1,)), ((), ())),
                         preferred_element_type=jnp.float32) + b2_ref[...]

    @pl.when(p == 2)
    def _():
        acc_ref[4:5, :] += jnp.sum(z3, axis=0)[None]
        acc_ref[5:6, :] += jnp.sum(z3 * z3, axis=0)[None]

    @pl.when(p < 3)
    def _():
        out_ref[...] = jnp.zeros((CT, C3), jnp.float32)

    @pl.when(p == 3)
    def _():
        m2 = acc_ref[4:5, :] / MF
        v2 = acc_ref[5:6, :] / MF - m2 * m2
        y3 = jnp.maximum((z3 - m2) / jnp.sqrt(v2 + EPS) * g2_ref[...]
                         + be2_ref[...], 0.0)
        out_ref[...] = jnp.max(y3.reshape(CT, NS, C3), axis=1)


def _mlp(z0, nxyz, wx, w1, w2, b1r, b2r, g0r, be0r, g1r, be1r, g2r, be2r,
         *, interpret=False):
    vec64 = pl.BlockSpec((1, C1), lambda p, t: (0, 0))
    vec128 = pl.BlockSpec((1, C3), lambda p, t: (0, 0))
    return pl.pallas_call(
        _mlp_body,
        grid=(4, N_TILES),
        in_specs=[
            pl.BlockSpec((M_TILE, C1), lambda p, t: (t, 0)),
            pl.BlockSpec((CT, B, 3), lambda p, t: (t % S_CH, 0, 0)),
            pl.BlockSpec((C1, 3), lambda p, t: (0, 0)),
            pl.BlockSpec((C2, C1), lambda p, t: (0, 0)),
            pl.BlockSpec((C3, C2), lambda p, t: (0, 0)),
            vec64, vec128, vec64, vec64, vec64, vec64, vec128, vec128,
        ],
        out_specs=pl.BlockSpec((CT, C3), lambda p, t: (t, 0)),
        out_shape=jax.ShapeDtypeStruct((B * S, C3), jnp.float32),
        scratch_shapes=[pltpu.VMEM((8, 128), jnp.float32)],
        interpret=interpret,
    )(z0, nxyz, wx, w1, w2, b1r, b2r, g0r, be0r, g1r, be1r, g2r, be2r)


def kernel(xyz, points, W0, b0, gamma0, beta0, W1, b1, gamma1, beta1,
           W2, b2, gamma2, beta2):
    wx = W0[:, :3]
    wp = W0[:, 3:]
    b0r = b0[None, :]
    nxyz = _fps(xyz)
    u_flat = _compute_u(xyz, points, wx, wp, b0r)
    gidx = _ball_query(xyz, nxyz)
    z0 = _make_gather()(u_flat, gidx.reshape(-1))
    new_points = _mlp(
        z0, nxyz, wx, W1, W2, b1[None, :], b2[None, :],
        gamma0[None, :], beta0[None, :], gamma1[None, :], beta1[None, :],
        gamma2[None, :], beta2[None, :])
    new_xyz_out = jnp.transpose(nxyz, (1, 2, 0))
    return new_xyz_out, new_points


# speedup vs baseline: 15.1125x; 1.5358x over previous
import functools

import jax
import jax.numpy as jnp
import numpy as np
from jax import lax
from jax.experimental import pallas as pl
from jax.experimental.pallas import tpu as pltpu
from jax.experimental.pallas import tpu_sc as plsc

B = 4
N = 8192
S = 1024
NS = 32
D_PTS = 64
C1 = 64
C2 = 64
C3 = 128
M_TOT = B * S * NS
EPS = 1e-5
R2 = float(np.float32(0.2) * np.float32(0.2))
BIG = N


FR = 64
FL = N // FR


def _fps_body(xyz_ref, nxyz_ref):
    x = xyz_ref[:, 0]
    y = xyz_ref[:, 1]
    z = xyz_ref[:, 2]
    nidx = (lax.broadcasted_iota(jnp.int32, (B, FR, FL), 1) * FL
            + lax.broadcasted_iota(jnp.int32, (B, FR, FL), 2))

    def body(i, st):
        dist, far = st
        oh = nidx == far
        cx = jnp.sum(jnp.where(oh, x, 0.0), axis=(1, 2), keepdims=True)
        cy = jnp.sum(jnp.where(oh, y, 0.0), axis=(1, 2), keepdims=True)
        cz = jnp.sum(jnp.where(oh, z, 0.0), axis=(1, 2), keepdims=True)
        cent = jnp.concatenate([cx, cy, cz], axis=2)
        nxyz_ref[pl.ds(i, 1)] = cent.reshape(1, B, 3)
        dx = x - cx
        dy = y - cy
        dz = z - cz
        d = (dx * dx + dy * dy) + dz * dz
        dist = jnp.minimum(dist, d)
        mx = jnp.max(dist, axis=(1, 2), keepdims=True)
        far = jnp.min(jnp.where(dist == mx, nidx, N), axis=(1, 2),
                      keepdims=True)
        return dist, far

    st = (jnp.full((B, FR, FL), 1e10, jnp.float32),
          jnp.zeros((B, 1, 1), jnp.int32))
    lax.fori_loop(0, S, body, st)


def _fps(xyz, *, interpret=False):
    return pl.pallas_call(
        _fps_body,
        out_shape=jax.ShapeDtypeStruct((S, B, 3), jnp.float32),
        interpret=interpret,
    )(xyz.reshape(B, 3, FR, FL))


N_CHUNK = 1024
N_CH = N // N_CHUNK

def _u_body(xyz_ref, pts_ref, wx_ref, wp_ref, b0_ref, u_ref):
    x = xyz_ref[0]
    p = pts_ref[0]
    wx = wx_ref[...]
    wp = wp_ref[...]
    u = lax.dot_general(x, wx, (((0,), (1,)), ((), ())),
                        preferred_element_type=jnp.float32)
    u = u + lax.dot_general(p, wp, (((0,), (1,)), ((), ())),
                            preferred_element_type=jnp.float32)
    u_ref[...] = u + b0_ref[...]


def _compute_u(xyz, points, wx, wp, b0r, *, interpret=False):
    return pl.pallas_call(
        _u_body,
        grid=(B, N_CH),
        in_specs=[
            pl.BlockSpec((1, 3, N_CHUNK), lambda b, c: (b, 0, c)),
            pl.BlockSpec((1, D_PTS, N_CHUNK), lambda b, c: (b, 0, c)),
            pl.BlockSpec((C1, 3), lambda b, c: (0, 0)),
            pl.BlockSpec((C1, D_PTS), lambda b, c: (0, 0)),
            pl.BlockSpec((1, C1), lambda b, c: (0, 0)),
        ],
        out_specs=pl.BlockSpec((N_CHUNK, C1), lambda b, c: (b * N_CH + c, 0)),
        out_shape=jax.ShapeDtypeStruct((B * N, C1), jnp.float32),
        interpret=interpret,
    )(xyz, points, wx, wp, b0r)


S_TILE = 128
S_CH = S // S_TILE

def _bq_body(xyz_ref, nxyz_ref, out_ref):
    b = pl.program_id(0)
    x = xyz_ref[0]
    bsel = (lax.broadcasted_iota(jnp.int32, (1, B, 1), 1) == b)
    nx = jnp.sum(jnp.where(bsel, nxyz_ref[...], 0.0), axis=1)
    sqx = jnp.sum(x * x, axis=0)[None, :]
    sqn = jnp.sum(nx * nx, axis=1)[:, None]
    dot = lax.dot_general(nx, x, (((1,), (0,)), ((), ())),
                          preferred_element_type=jnp.float32)
    d = (sqn + sqx) - 2.0 * dot
    iota = lax.broadcasted_iota(jnp.int32, (S_TILE, N), 1)
    keys = jnp.where(d <= R2, iota, BIG)
    cols = []
    first = None
    for _ in range(NS):
        m = jnp.min(keys, axis=1)
        if first is None:
            first = m
        keys = jnp.where(keys == m[:, None], BIG, keys)
        cols.append(m)
    sel = jnp.stack(cols, axis=1)
    sel = jnp.where(sel == BIG, first[:, None], sel)
    out_ref[...] = sel + b * N


def _ball_query(xyz, nxyz, *, interpret=False):
    return pl.pallas_call(
        _bq_body,
        grid=(B, S_CH),
        in_specs=[
            pl.BlockSpec((1, 3, N), lambda b, c: (b, 0, 0)),
            pl.BlockSpec((S_TILE, B, 3), lambda b, c: (c, 0, 0)),
        ],
        out_specs=pl.BlockSpec((S_TILE, NS), lambda b, c: (b * S_CH + c, 0)),
        out_shape=jax.ShapeDtypeStruct((B * S, NS), jnp.int32),
        interpret=interpret,
    )(xyz, nxyz)


GCHUNK = 128
PER_W = None

@functools.lru_cache(maxsize=1)
def _make_gather():
    nw = 32
    per_w = M_TOT // nw
    iters = per_w // GCHUNK
    mesh = plsc.VectorSubcoreMesh(core_axis_name="c", subcore_axis_name="s")

    @functools.partial(
        pl.kernel,
        mesh=mesh,
        compiler_params=pltpu.CompilerParams(use_tc_tiling_on_sc=False),
        out_type=jax.ShapeDtypeStruct((M_TOT, C1), jnp.float32),
        scratch_types=[
            pltpu.VMEM((GCHUNK,), jnp.int32),
            pltpu.VMEM((GCHUNK, C1), jnp.float32),
            pltpu.SemaphoreType.DMA,
        ],
    )
    def gather_k(table_hbm, idx_hbm, out_hbm, idx_v, rows_v, sem):
        c = lax.axis_index("c")
        s = lax.axis_index("s")
        wid = s * 2 + c
        base = wid * per_w

        def body(i, carry):
            off = base + i * GCHUNK
            pltpu.sync_copy(idx_hbm.at[pl.ds(off, GCHUNK)], idx_v)
            pltpu.async_copy(table_hbm.at[idx_v], rows_v, sem).wait()
            pltpu.sync_copy(rows_v, out_hbm.at[pl.ds(off, GCHUNK)])
            return carry

        lax.fori_loop(0, iters, body, 0)

    return gather_k


M_TILE = 4096
CT = M_TILE // NS
N_TILES = M_TOT // M_TILE
MF = float(M_TOT)

def _mlp_body(z0_ref, nxyz_ref, wx_ref, w1_ref, w2_ref,
              b1_ref, b2_ref, g0_ref, be0_ref, g1_ref, be1_ref,
              g2_ref, be2_ref, out_ref, acc_ref):
    p = pl.program_id(0)
    t = pl.program_id(1)

    @pl.when((p == 0) & (t == 0))
    def _():
        acc_ref[...] = jnp.zeros((8, 128), jnp.float32)

    bsel = (lax.broadcasted_iota(jnp.int32, (1, B, 1), 1) == t // S_CH)
    nx = jnp.sum(jnp.where(bsel, nxyz_ref[...], 0.0), axis=1)
    v = lax.dot_general(nx, wx_ref[...], (((1,), (1,)), ((), ())),
                        preferred_element_type=jnp.float32)
    r_iota = lax.broadcasted_iota(jnp.int32, (M_TILE, CT), 0) // NS
    c_iota = lax.broadcasted_iota(jnp.int32, (M_TILE, CT), 1)
    expand = (r_iota == c_iota).astype(jnp.float32)
    vexp = lax.dot_general(expand, v, (((1,), (0,)), ((), ())),
                           preferred_element_type=jnp.float32)
    z = z0_ref[...] - vexp

    @pl.when(p == 0)
    def _():
        acc_ref[0:1, :C1] += jnp.sum(z, axis=0)[None]
        acc_ref[1:2, :C1] += jnp.sum(z * z, axis=0)[None]

    m0 = acc_ref[0:1, :C1] / MF
    v0 = acc_ref[1:2, :C1] / MF - m0 * m0
    y1 = jnp.maximum((z - m0) / jnp.sqrt(v0 + EPS) * g0_ref[...]
                     + be0_ref[...], 0.0)
    z2 = lax.dot_general(y1, w1_ref[...], (((1,), (1,)), ((), ())),
                         preferred_element_type=jnp.float32) + b1_ref[...]

    @pl.when(p == 1)
    def _():
        acc_ref[2:3, :C1] += jnp.sum(z2, axis=0)[None]
        acc_ref[3:4, :C1] += jnp.sum(z2 * z2, axis=0)[None]

    m1 = acc_ref[2:3, :C1] / MF
    v1 = acc_ref[3:4, :C1] / MF - m1 * m1
    y2 = jnp.maximum((z2 - m1) / jnp.sqrt(v1 + EPS) * g1_ref[...]
                     + be1_ref[...], 0.0)
    z3 = lax.dot_general(y2, w2_ref[...], (((1,), (1,)), ((), ())),
                         preferred_element_type=jnp.float32) + b2_ref[...]

    @pl.when(p == 2)
    def _():
        acc_ref[4:5, :] += jnp.sum(z3, axis=0)[None]
        acc_ref[5:6, :] += jnp.sum(z3 * z3, axis=0)[None]

    @pl.when(p < 3)
    def _():
        out_ref[...] = jnp.zeros((CT, C3), jnp.float32)

    @pl.when(p == 3)
    def _():
        m2 = acc_ref[4:5, :] / MF
        v2 = acc_ref[5:6, :] / MF - m2 * m2
        y3 = jnp.maximum((z3 - m2) / jnp.sqrt(v2 + EPS) * g2_ref[...]
                         + be2_ref[...], 0.0)
        out_ref[...] = jnp.max(y3.reshape(CT, NS, C3), axis=1)


def _mlp(z0, nxyz, wx, w1, w2, b1r, b2r, g0r, be0r, g1r, be1r, g2r, be2r,
         *, interpret=False):
    vec64 = pl.BlockSpec((1, C1), lambda p, t: (0, 0))
    vec128 = pl.BlockSpec((1, C3), lambda p, t: (0, 0))
    return pl.pallas_call(
        _mlp_body,
        grid=(4, N_TILES),
        in_specs=[
            pl.BlockSpec((M_TILE, C1), lambda p, t: (t, 0)),
            pl.BlockSpec((CT, B, 3), lambda p, t: (t % S_CH, 0, 0)),
            pl.BlockSpec((C1, 3), lambda p, t: (0, 0)),
            pl.BlockSpec((C2, C1), lambda p, t: (0, 0)),
            pl.BlockSpec((C3, C2), lambda p, t: (0, 0)),
            vec64, vec128, vec64, vec64, vec64, vec64, vec128, vec128,
        ],
        out_specs=pl.BlockSpec((CT, C3), lambda p, t: (t, 0)),
        out_shape=jax.ShapeDtypeStruct((B * S, C3), jnp.float32),
        scratch_shapes=[pltpu.VMEM((8, 128), jnp.float32)],
        interpret=interpret,
    )(z0, nxyz, wx, w1, w2, b1r, b2r, g0r, be0r, g1r, be1r, g2r, be2r)


def kernel(xyz, points, W0, b0, gamma0, beta0, W1, b1, gamma1, beta1,
           W2, b2, gamma2, beta2):
    wx = W0[:, :3]
    wp = W0[:, 3:]
    b0r = b0[None, :]
    nxyz = _fps(xyz)
    u_flat = _compute_u(xyz, points, wx, wp, b0r)
    gidx = _ball_query(xyz, nxyz)
    z0 = _make_gather()(u_flat, gidx.reshape(-1))
    new_points = _mlp(
        z0, nxyz, wx, W1, W2, b1[None, :], b2[None, :],
        gamma0[None, :], beta0[None, :], gamma1[None, :], beta1[None, :],
        gamma2[None, :], beta2[None, :])
    new_xyz_out = jnp.transpose(nxyz, (1, 2, 0))
    return new_xyz_out, new_points


# speedup vs baseline: 15.9951x; 1.0584x over previous
import functools

import jax
import jax.numpy as jnp
import numpy as np
from jax import lax
from jax.experimental import pallas as pl
from jax.experimental.pallas import tpu as pltpu
from jax.experimental.pallas import tpu_sc as plsc

B = 4
N = 8192
S = 1024
NS = 32
D_PTS = 64
C1 = 64
C2 = 64
C3 = 128
M_TOT = B * S * NS
EPS = 1e-5
R2 = float(np.float32(0.2) * np.float32(0.2))
BIG = N


FR = 64
FL = N // FR


def _fps_body(xyz_ref, nxyz_ref):
    x = xyz_ref[:, 0]
    y = xyz_ref[:, 1]
    z = xyz_ref[:, 2]
    nidx = (lax.broadcasted_iota(jnp.int32, (B, FR, FL), 1) * FL
            + lax.broadcasted_iota(jnp.int32, (B, FR, FL), 2))

    def body(i, st):
        dist, far = st
        oh = nidx == far
        cx = jnp.sum(jnp.where(oh, x, 0.0), axis=(1, 2), keepdims=True)
        cy = jnp.sum(jnp.where(oh, y, 0.0), axis=(1, 2), keepdims=True)
        cz = jnp.sum(jnp.where(oh, z, 0.0), axis=(1, 2), keepdims=True)
        cent = jnp.concatenate([cx, cy, cz], axis=2)
        nxyz_ref[pl.ds(i, 1)] = cent.reshape(1, B, 3)
        dx = x - cx
        dy = y - cy
        dz = z - cz
        d = (dx * dx + dy * dy) + dz * dz
        dist = jnp.minimum(dist, d)
        mx = jnp.max(dist, axis=(1, 2), keepdims=True)
        far = jnp.min(jnp.where(dist == mx, nidx, N), axis=(1, 2),
                      keepdims=True)
        return dist, far

    st = (jnp.full((B, FR, FL), 1e10, jnp.float32),
          jnp.zeros((B, 1, 1), jnp.int32))
    lax.fori_loop(0, S, body, st)


def _fps(xyz, *, interpret=False):
    return pl.pallas_call(
        _fps_body,
        out_shape=jax.ShapeDtypeStruct((S, B, 3), jnp.float32),
        interpret=interpret,
    )(xyz.reshape(B, 3, FR, FL))


N_CHUNK = 1024
N_CH = N // N_CHUNK

def _u_body(xyz_ref, pts_ref, wx_ref, wp_ref, b0_ref, u_ref):
    x = xyz_ref[0]
    p = pts_ref[0]
    wx = wx_ref[...]
    wp = wp_ref[...]
    u = lax.dot_general(x, wx, (((0,), (1,)), ((), ())),
                        preferred_element_type=jnp.float32)
    u = u + lax.dot_general(p, wp, (((0,), (1,)), ((), ())),
                            preferred_element_type=jnp.float32)
    u_ref[...] = u + b0_ref[...]


def _compute_u(xyz, points, wx, wp, b0r, *, interpret=False):
    return pl.pallas_call(
        _u_body,
        grid=(B, N_CH),
        in_specs=[
            pl.BlockSpec((1, 3, N_CHUNK), lambda b, c: (b, 0, c)),
            pl.BlockSpec((1, D_PTS, N_CHUNK), lambda b, c: (b, 0, c)),
            pl.BlockSpec((C1, 3), lambda b, c: (0, 0)),
            pl.BlockSpec((C1, D_PTS), lambda b, c: (0, 0)),
            pl.BlockSpec((1, C1), lambda b, c: (0, 0)),
        ],
        out_specs=pl.BlockSpec((N_CHUNK, C1), lambda b, c: (b * N_CH + c, 0)),
        out_shape=jax.ShapeDtypeStruct((B * N, C1), jnp.float32),
        interpret=interpret,
    )(xyz, points, wx, wp, b0r)


S_TILE = 128
S_CH = S // S_TILE

def _bq_body(xyz_ref, nxyz_ref, out_ref):
    b = pl.program_id(0)
    x = xyz_ref[0]
    bsel = (lax.broadcasted_iota(jnp.int32, (1, B, 1), 1) == b)
    nx = jnp.sum(jnp.where(bsel, nxyz_ref[...], 0.0), axis=1)
    sqx = jnp.sum(x * x, axis=0)[None, :]
    sqn = jnp.sum(nx * nx, axis=1)[:, None]
    dot = lax.dot_general(nx, x, (((1,), (0,)), ((), ())),
                          preferred_element_type=jnp.float32)
    d = (sqn + sqx) - 2.0 * dot
    iota = lax.broadcasted_iota(jnp.int32, (S_TILE, N), 1)
    keys = jnp.where(d <= R2, iota, BIG)
    cols = []
    first = None
    for _ in range(NS):
        m = jnp.min(keys, axis=1)
        if first is None:
            first = m
        keys = jnp.where(keys == m[:, None], BIG, keys)
        cols.append(m)
    sel = jnp.stack(cols, axis=1)
    sel = jnp.where(sel == BIG, first[:, None], sel)
    out_ref[...] = sel + b * N


def _ball_query(xyz, nxyz, *, interpret=False):
    return pl.pallas_call(
        _bq_body,
        grid=(B, S_CH),
        in_specs=[
            pl.BlockSpec((1, 3, N), lambda b, c: (b, 0, 0)),
            pl.BlockSpec((S_TILE, B, 3), lambda b, c: (c, 0, 0)),
        ],
        out_specs=pl.BlockSpec((S_TILE, NS), lambda b, c: (b * S_CH + c, 0)),
        out_shape=jax.ShapeDtypeStruct((B * S, NS), jnp.int32),
        interpret=interpret,
    )(xyz, nxyz)


GCHUNK = 128
PER_W = None

@functools.lru_cache(maxsize=1)
def _make_gather():
    nw = 32
    per_w = M_TOT // nw
    iters = per_w // GCHUNK
    mesh = plsc.VectorSubcoreMesh(core_axis_name="c", subcore_axis_name="s")

    @functools.partial(
        pl.kernel,
        mesh=mesh,
        compiler_params=pltpu.CompilerParams(use_tc_tiling_on_sc=False),
        out_type=jax.ShapeDtypeStruct((M_TOT, C1), jnp.float32),
        scratch_types=[
            pltpu.VMEM((GCHUNK,), jnp.int32),
            pltpu.VMEM((GCHUNK, C1), jnp.float32),
            pltpu.SemaphoreType.DMA,
        ],
    )
    def gather_k(table_hbm, idx_hbm, out_hbm, idx_v, rows_v, sem):
        c = lax.axis_index("c")
        s = lax.axis_index("s")
        wid = s * 2 + c
        base = wid * per_w

        def body(i, carry):
            off = base + i * GCHUNK
            pltpu.sync_copy(idx_hbm.at[pl.ds(off, GCHUNK)], idx_v)
            pltpu.async_copy(table_hbm.at[idx_v], rows_v, sem).wait()
            pltpu.sync_copy(rows_v, out_hbm.at[pl.ds(off, GCHUNK)])
            return carry

        lax.fori_loop(0, iters, body, 0)

    return gather_k


M_TILE = 4096
MT_D = M_TILE // 2
CT = M_TILE // NS
GRP = MT_D // CT
N_TILES = M_TOT // M_TILE
MF = float(M_TOT)

def _mlp_body(z0_ref, nxyz_ref, wx_ref, w1d_ref, w2d_ref,
              b1d_ref, b2d_ref, g0d_ref, be0d_ref, g1d_ref, be1d_ref,
              g2d_ref, be2d_ref, out_ref, acc_ref):
    p = pl.program_id(0)
    t = pl.program_id(1)

    @pl.when((p == 0) & (t == 0))
    def _():
        acc_ref[...] = jnp.zeros((8, 2 * C3), jnp.float32)

    def fold64(row):
        s = row[:, :C1] + row[:, C1:]
        return jnp.concatenate([s, s], axis=1)

    def fold128(row):
        s = row[:, :C3] + row[:, C3:]
        return jnp.concatenate([s, s], axis=1)

    bsel = (lax.broadcasted_iota(jnp.int32, (1, B, 1), 1) == t // S_CH)
    nx = jnp.sum(jnp.where(bsel, nxyz_ref[...], 0.0), axis=1)
    v = lax.dot_general(nx, wx_ref[...], (((1,), (1,)), ((), ())),
                        preferred_element_type=jnp.float32)
    vd = jnp.concatenate([v, v], axis=1)
    r_iota = lax.broadcasted_iota(jnp.int32, (MT_D, CT), 0) // GRP
    c_iota = lax.broadcasted_iota(jnp.int32, (MT_D, CT), 1)
    expand = (r_iota == c_iota).astype(jnp.float32)
    vexp = lax.dot_general(expand, vd, (((1,), (0,)), ((), ())),
                           preferred_element_type=jnp.float32)
    z = z0_ref[...] - vexp

    @pl.when(p == 0)
    def _():
        acc_ref[0:1, :2 * C1] += jnp.sum(z, axis=0)[None]
        acc_ref[1:2, :2 * C1] += jnp.sum(z * z, axis=0)[None]

    m0 = fold64(acc_ref[0:1, :2 * C1]) / MF
    v0 = fold64(acc_ref[1:2, :2 * C1]) / MF - m0 * m0
    a0 = g0d_ref[...] / jnp.sqrt(v0 + EPS)
    c0 = be0d_ref[...] - m0 * a0
    y1 = jnp.maximum(z * a0 + c0, 0.0)
    z2 = lax.dot_general(y1, w1d_ref[...], (((1,), (0,)), ((), ())),
                         preferred_element_type=jnp.float32) + b1d_ref[...]

    @pl.when(p == 1)
    def _():
        acc_ref[2:3, :2 * C1] += jnp.sum(z2, axis=0)[None]
        acc_ref[3:4, :2 * C1] += jnp.sum(z2 * z2, axis=0)[None]

    m1 = fold64(acc_ref[2:3, :2 * C1]) / MF
    v1 = fold64(acc_ref[3:4, :2 * C1]) / MF - m1 * m1
    a1 = g1d_ref[...] / jnp.sqrt(v1 + EPS)
    c1 = be1d_ref[...] - m1 * a1
    y2 = jnp.maximum(z2 * a1 + c1, 0.0)
    z3 = lax.dot_general(y2, w2d_ref[...], (((1,), (0,)), ((), ())),
                         preferred_element_type=jnp.float32) + b2d_ref[...]

    @pl.when(p == 2)
    def _():
        acc_ref[4:5, :] += jnp.sum(z3, axis=0)[None]
        acc_ref[5:6, :] += jnp.sum(z3 * z3, axis=0)[None]

    @pl.when(p < 3)
    def _():
        out_ref[...] = jnp.zeros((CT, C3), jnp.float32)

    @pl.when(p == 3)
    def _():
        m2 = fold128(acc_ref[4:5, :]) / MF
        v2 = fold128(acc_ref[5:6, :]) / MF - m2 * m2
        a2 = g2d_ref[...] / jnp.sqrt(v2 + EPS)
        c2 = be2d_ref[...] - m2 * a2
        y3 = jnp.maximum(z3 * a2 + c2, 0.0)
        pooled = jnp.max(y3.reshape(CT, GRP, 2 * C3), axis=1)
        out_ref[...] = jnp.maximum(pooled[:, :C3], pooled[:, C3:])


def _mlp(z0d, nxyz, wx, w1d, w2d, b1d, b2d, g0d, be0d, g1d, be1d, g2d, be2d,
         *, interpret=False):
    vec128 = pl.BlockSpec((1, 2 * C1), lambda p, t: (0, 0))
    vec256 = pl.BlockSpec((1, 2 * C3), lambda p, t: (0, 0))
    return pl.pallas_call(
        _mlp_body,
        grid=(4, N_TILES),
        in_specs=[
            pl.BlockSpec((MT_D, 2 * C1), lambda p, t: (t, 0)),
            pl.BlockSpec((CT, B, 3), lambda p, t: (t % S_CH, 0, 0)),
            pl.BlockSpec((C1, 3), lambda p, t: (0, 0)),
            pl.BlockSpec((2 * C1, 2 * C2), lambda p, t: (0, 0)),
            pl.BlockSpec((2 * C2, 2 * C3), lambda p, t: (0, 0)),
            vec128, vec256, vec128, vec128, vec128, vec128, vec256, vec256,
        ],
        out_specs=pl.BlockSpec((CT, C3), lambda p, t: (t, 0)),
        out_shape=jax.ShapeDtypeStruct((B * S, C3), jnp.float32),
        scratch_shapes=[pltpu.VMEM((8, 2 * C3), jnp.float32)],
        interpret=interpret,
    )(z0d, nxyz, wx, w1d, w2d, b1d, b2d, g0d, be0d, g1d, be1d, g2d, be2d)


def _doubled_params(W1, W2, b1, b2, gamma0, beta0, gamma1, beta1,
                    gamma2, beta2):
    z64 = jnp.zeros((C1, C2), jnp.float32)
    z128 = jnp.zeros((C2, C3), jnp.float32)
    w1d = jnp.block([[W1.T, z64], [z64, W1.T]])
    w2d = jnp.block([[W2.T, z128], [z128, W2.T]])
    dbl = lambda a: jnp.concatenate([a, a])[None, :]
    return (w1d, w2d, dbl(b1), dbl(b2), dbl(gamma0), dbl(beta0),
            dbl(gamma1), dbl(beta1), dbl(gamma2), dbl(beta2))


def kernel(xyz, points, W0, b0, gamma0, beta0, W1, b1, gamma1, beta1,
           W2, b2, gamma2, beta2):
    wx = W0[:, :3]
    wp = W0[:, 3:]
    b0r = b0[None, :]
    nxyz = _fps(xyz)
    u_flat = _compute_u(xyz, points, wx, wp, b0r)
    gidx = _ball_query(xyz, nxyz)
    z0 = _make_gather()(u_flat, gidx.reshape(-1))
    z0d = z0.reshape(M_TOT // 2, 2 * C1)
    dparams = _doubled_params(W1, W2, b1, b2, gamma0, beta0, gamma1, beta1,
                              gamma2, beta2)
    new_points = _mlp(z0d, nxyz, wx, *dparams)
    new_xyz_out = jnp.transpose(nxyz, (1, 2, 0))
    return new_xyz_out, new_points


# speedup vs baseline: 18.2571x; 1.1414x over previous
import functools

import jax
import jax.numpy as jnp
import numpy as np
from jax import lax
from jax.experimental import pallas as pl
from jax.experimental.pallas import tpu as pltpu
from jax.experimental.pallas import tpu_sc as plsc

B = 4
N = 8192
S = 1024
NS = 32
D_PTS = 64
C1 = 64
C2 = 64
C3 = 128
M_TOT = B * S * NS
EPS = 1e-5
R2 = float(np.float32(0.2) * np.float32(0.2))
BIG = N


FR = 64
FL = N // FR


def _fps_body(xyz_ref, nxyz_ref):
    x = xyz_ref[:, 0]
    y = xyz_ref[:, 1]
    z = xyz_ref[:, 2]
    nidx = (lax.broadcasted_iota(jnp.int32, (B, FR, FL), 1) * FL
            + lax.broadcasted_iota(jnp.int32, (B, FR, FL), 2))

    def body(i, st):
        dist, far = st
        oh = nidx == far
        cx = jnp.sum(jnp.where(oh, x, 0.0), axis=(1, 2), keepdims=True)
        cy = jnp.sum(jnp.where(oh, y, 0.0), axis=(1, 2), keepdims=True)
        cz = jnp.sum(jnp.where(oh, z, 0.0), axis=(1, 2), keepdims=True)
        cent = jnp.concatenate([cx, cy, cz], axis=2)
        nxyz_ref[pl.ds(i, 1)] = cent.reshape(1, B, 3)
        dx = x - cx
        dy = y - cy
        dz = z - cz
        d = (dx * dx + dy * dy) + dz * dz
        dist = jnp.minimum(dist, d)
        mx = jnp.max(dist, axis=(1, 2), keepdims=True)
        far = jnp.min(jnp.where(dist == mx, nidx, N), axis=(1, 2),
                      keepdims=True)
        return dist, far

    st = (jnp.full((B, FR, FL), 1e10, jnp.float32),
          jnp.zeros((B, 1, 1), jnp.int32))
    lax.fori_loop(0, S, body, st)


def _fps(xyz, *, interpret=False):
    return pl.pallas_call(
        _fps_body,
        out_shape=jax.ShapeDtypeStruct((S, B, 3), jnp.float32),
        interpret=interpret,
    )(xyz.reshape(B, 3, FR, FL))


N_CHUNK = 1024
N_CH = N // N_CHUNK

def _u_body(xyz_ref, pts_ref, wx_ref, wp_ref, b0_ref, u_ref):
    x = xyz_ref[0]
    p = pts_ref[0]
    wx = wx_ref[...]
    wp = wp_ref[...]
    u = lax.dot_general(x, wx, (((0,), (1,)), ((), ())),
                        preferred_element_type=jnp.float32)
    u = u + lax.dot_general(p, wp, (((0,), (1,)), ((), ())),
                            preferred_element_type=jnp.float32)
    u_ref[...] = u + b0_ref[...]


def _compute_u(xyz, points, wx, wp, b0r, *, interpret=False):
    return pl.pallas_call(
        _u_body,
        grid=(B, N_CH),
        in_specs=[
            pl.BlockSpec((1, 3, N_CHUNK), lambda b, c: (b, 0, c)),
            pl.BlockSpec((1, D_PTS, N_CHUNK), lambda b, c: (b, 0, c)),
            pl.BlockSpec((C1, 3), lambda b, c: (0, 0)),
            pl.BlockSpec((C1, D_PTS), lambda b, c: (0, 0)),
            pl.BlockSpec((1, C1), lambda b, c: (0, 0)),
        ],
        out_specs=pl.BlockSpec((N_CHUNK, C1), lambda b, c: (b * N_CH + c, 0)),
        out_shape=jax.ShapeDtypeStruct((B * N, C1), jnp.float32),
        interpret=interpret,
    )(xyz, points, wx, wp, b0r)


S_TILE = 128
S_CH = S // S_TILE

def _bq_body(xyz_ref, nxyz_ref, out_ref):
    b = pl.program_id(0)
    x = xyz_ref[0]
    bsel = (lax.broadcasted_iota(jnp.int32, (1, B, 1), 1) == b)
    nx = jnp.sum(jnp.where(bsel, nxyz_ref[...], 0.0), axis=1)
    sqx = jnp.sum(x * x, axis=0)[None, :]
    sqn = jnp.sum(nx * nx, axis=1)[:, None]
    dot = lax.dot_general(nx, x, (((1,), (0,)), ((), ())),
                          preferred_element_type=jnp.float32)
    d = (sqn + sqx) - 2.0 * dot
    iota = lax.broadcasted_iota(jnp.int32, (S_TILE, N), 1)
    keys = jnp.where(d <= R2, iota, BIG)
    cols = []
    first = None
    for _ in range(NS):
        m = jnp.min(keys, axis=1)
        if first is None:
            first = m
        keys = jnp.where(keys == m[:, None], BIG, keys)
        cols.append(m)
    sel = jnp.stack(cols, axis=1)
    sel = jnp.where(sel == BIG, first[:, None], sel)
    out_ref[...] = sel + b * N


def _ball_query(xyz, nxyz, *, interpret=False):
    return pl.pallas_call(
        _bq_body,
        grid=(B, S_CH),
        in_specs=[
            pl.BlockSpec((1, 3, N), lambda b, c: (b, 0, 0)),
            pl.BlockSpec((S_TILE, B, 3), lambda b, c: (c, 0, 0)),
        ],
        out_specs=pl.BlockSpec((S_TILE, NS), lambda b, c: (b * S_CH + c, 0)),
        out_shape=jax.ShapeDtypeStruct((B * S, NS), jnp.int32),
        interpret=interpret,
    )(xyz, nxyz)


GCHUNK = 128
PER_W = None

@functools.lru_cache(maxsize=1)
def _make_gather():
    nw = 32
    per_w = M_TOT // nw
    iters = per_w // GCHUNK
    mesh = plsc.VectorSubcoreMesh(core_axis_name="c", subcore_axis_name="s")

    @functools.partial(
        pl.kernel,
        mesh=mesh,
        compiler_params=pltpu.CompilerParams(use_tc_tiling_on_sc=False),
        out_type=jax.ShapeDtypeStruct((M_TOT, C1), jnp.float32),
        scratch_types=[
            pltpu.VMEM((GCHUNK,), jnp.int32),
            pltpu.VMEM((GCHUNK, C1), jnp.float32),
            pltpu.SemaphoreType.DMA,
        ],
    )
    def gather_k(table_hbm, idx_hbm, out_hbm, idx_v, rows_v, sem):
        c = lax.axis_index("c")
        s = lax.axis_index("s")
        wid = s * 2 + c
        base = wid * per_w

        def body(i, carry):
            off = base + i * GCHUNK
            pltpu.sync_copy(idx_hbm.at[pl.ds(off, GCHUNK)], idx_v)
            pltpu.async_copy(table_hbm.at[idx_v], rows_v, sem).wait()
            pltpu.sync_copy(rows_v, out_hbm.at[pl.ds(off, GCHUNK)])
            return carry

        lax.fori_loop(0, iters, body, 0)

    return gather_k


M_TILE = 4096
MT_D = M_TILE // 2
CT = M_TILE // NS
GRP = MT_D // CT
N_TILES = M_TOT // M_TILE
MF = float(M_TOT)

def _mlp_body(z0_ref, nxyz_ref, wx_ref, w1d_ref, w2d_ref,
              b1d_ref, b2d_ref, g0d_ref, be0d_ref, g1d_ref, be1d_ref,
              g2d_ref, be2d_ref, out_ref, acc_ref, zbuf_ref):
    p = pl.program_id(0)
    t = pl.program_id(1)
    zs = pl.ds(t * MT_D, MT_D)

    @pl.when((p == 0) & (t == 0))
    def _():
        acc_ref[...] = jnp.zeros((8, 2 * C3), jnp.float32)

    def fold64(row):
        s = row[:, :C1] + row[:, C1:]
        return jnp.concatenate([s, s], axis=1)

    def fold128(row):
        s = row[:, :C3] + row[:, C3:]
        return jnp.concatenate([s, s], axis=1)

    def bn1_coeffs():
        m1 = fold64(acc_ref[2:3, :2 * C1]) / MF
        v1 = fold64(acc_ref[3:4, :2 * C1]) / MF - m1 * m1
        a1 = g1d_ref[...] / jnp.sqrt(v1 + EPS)
        return a1, be1d_ref[...] - m1 * a1

    def z3_of(z2):
        a1, c1 = bn1_coeffs()
        y2 = jnp.maximum(z2 * a1 + c1, 0.0)
        return lax.dot_general(y2, w2d_ref[...], (((1,), (0,)), ((), ())),
                               preferred_element_type=jnp.float32
                               ) + b2d_ref[...]

    @pl.when(p == 0)
    def _():
        bsel = (lax.broadcasted_iota(jnp.int32, (1, B, 1), 1) == t // S_CH)
        nx = jnp.sum(jnp.where(bsel, nxyz_ref[...], 0.0), axis=1)
        v = lax.dot_general(nx, wx_ref[...], (((1,), (1,)), ((), ())),
                            preferred_element_type=jnp.float32)
        vd = jnp.concatenate([v, v], axis=1)
        r_iota = lax.broadcasted_iota(jnp.int32, (MT_D, CT), 0) // GRP
        c_iota = lax.broadcasted_iota(jnp.int32, (MT_D, CT), 1)
        expand = (r_iota == c_iota).astype(jnp.float32)
        vexp = lax.dot_general(expand, vd, (((1,), (0,)), ((), ())),
                               preferred_element_type=jnp.float32)
        z = z0_ref[...] - vexp
        zbuf_ref[zs, :] = z
        acc_ref[0:1, :2 * C1] += jnp.sum(z, axis=0)[None]
        acc_ref[1:2, :2 * C1] += jnp.sum(z * z, axis=0)[None]
        out_ref[...] = jnp.zeros((CT, C3), jnp.float32)

    @pl.when(p == 1)
    def _():
        z = zbuf_ref[zs, :]
        m0 = fold64(acc_ref[0:1, :2 * C1]) / MF
        v0 = fold64(acc_ref[1:2, :2 * C1]) / MF - m0 * m0
        a0 = g0d_ref[...] / jnp.sqrt(v0 + EPS)
        c0 = be0d_ref[...] - m0 * a0
        y1 = jnp.maximum(z * a0 + c0, 0.0)
        z2 = lax.dot_general(y1, w1d_ref[...], (((1,), (0,)), ((), ())),
                             preferred_element_type=jnp.float32) + b1d_ref[...]
        zbuf_ref[zs, :] = z2
        acc_ref[2:3, :2 * C1] += jnp.sum(z2, axis=0)[None]
        acc_ref[3:4, :2 * C1] += jnp.sum(z2 * z2, axis=0)[None]
        out_ref[...] = jnp.zeros((CT, C3), jnp.float32)

    @pl.when(p == 2)
    def _():
        z3 = z3_of(zbuf_ref[zs, :])
        acc_ref[4:5, :] += jnp.sum(z3, axis=0)[None]
        acc_ref[5:6, :] += jnp.sum(z3 * z3, axis=0)[None]
        out_ref[...] = jnp.zeros((CT, C3), jnp.float32)

    @pl.when(p == 3)
    def _():
        z3 = z3_of(zbuf_ref[zs, :])
        m2 = fold128(acc_ref[4:5, :]) / MF
        v2 = fold128(acc_ref[5:6, :]) / MF - m2 * m2
        a2 = g2d_ref[...] / jnp.sqrt(v2 + EPS)
        c2 = be2d_ref[...] - m2 * a2
        y3 = jnp.maximum(z3 * a2 + c2, 0.0)
        pooled = jnp.max(y3.reshape(CT, GRP, 2 * C3), axis=1)
        out_ref[...] = jnp.maximum(pooled[:, :C3], pooled[:, C3:])


def _mlp(z0d, nxyz, wx, w1d, w2d, b1d, b2d, g0d, be0d, g1d, be1d, g2d, be2d,
         *, interpret=False):
    vec128 = pl.BlockSpec((1, 2 * C1), lambda p, t: (0, 0))
    vec256 = pl.BlockSpec((1, 2 * C3), lambda p, t: (0, 0))
    return pl.pallas_call(
        _mlp_body,
        grid=(4, N_TILES),
        in_specs=[
            pl.BlockSpec((MT_D, 2 * C1),
                         lambda p, t: (jnp.where(p == 0, t, 0), 0)),
            pl.BlockSpec((CT, B, 3), lambda p, t: (t % S_CH, 0, 0)),
            pl.BlockSpec((C1, 3), lambda p, t: (0, 0)),
            pl.BlockSpec((2 * C1, 2 * C2), lambda p, t: (0, 0)),
            pl.BlockSpec((2 * C2, 2 * C3), lambda p, t: (0, 0)),
            vec128, vec256, vec128, vec128, vec128, vec128, vec256, vec256,
        ],
        out_specs=pl.BlockSpec((CT, C3), lambda p, t: (t, 0)),
        out_shape=jax.ShapeDtypeStruct((B * S, C3), jnp.float32),
        scratch_shapes=[pltpu.VMEM((8, 2 * C3), jnp.float32),
                        pltpu.VMEM((M_TOT // 2, 2 * C1), jnp.float32)],
        compiler_params=pltpu.CompilerParams(
            vmem_limit_bytes=50 * 1024 * 1024),
        interpret=interpret,
    )(z0d, nxyz, wx, w1d, w2d, b1d, b2d, g0d, be0d, g1d, be1d, g2d, be2d)


def _doubled_params(W1, W2, b1, b2, gamma0, beta0, gamma1, beta1,
                    gamma2, beta2):
    z64 = jnp.zeros((C1, C2), jnp.float32)
    z128 = jnp.zeros((C2, C3), jnp.float32)
    w1d = jnp.block([[W1.T, z64], [z64, W1.T]])
    w2d = jnp.block([[W2.T, z128], [z128, W2.T]])
    dbl = lambda a: jnp.concatenate([a, a])[None, :]
    return (w1d, w2d, dbl(b1), dbl(b2), dbl(gamma0), dbl(beta0),
            dbl(gamma1), dbl(beta1), dbl(gamma2), dbl(beta2))


def kernel(xyz, points, W0, b0, gamma0, beta0, W1, b1, gamma1, beta1,
           W2, b2, gamma2, beta2):
    wx = W0[:, :3]
    wp = W0[:, 3:]
    b0r = b0[None, :]
    nxyz = _fps(xyz)
    u_flat = _compute_u(xyz, points, wx, wp, b0r)
    gidx = _ball_query(xyz, nxyz)
    z0 = _make_gather()(u_flat, gidx.reshape(-1))
    z0d = z0.reshape(M_TOT // 2, 2 * C1)
    dparams = _doubled_params(W1, W2, b1, b2, gamma0, beta0, gamma1, beta1,
                              gamma2, beta2)
    new_points = _mlp(z0d, nxyz, wx, *dparams)
    new_xyz_out = jnp.transpose(nxyz, (1, 2, 0))
    return new_xyz_out, new_points
